# Initial kernel scaffold; baseline (speedup 1.0000x reference)
import jax
import jax.numpy as jnp
from jax.experimental import pallas as pl


def kernel(x, gate_weight, gate_up_proj, down_proj):
    raise NotImplementedError("write your pallas kernel here")


# speedup vs baseline: 1.5003x; 1.5003x over previous
import functools

import jax
import jax.numpy as jnp
from jax.experimental import pallas as pl
from jax.experimental.pallas import tpu as pltpu

HIDDEN = 1024
INTER = 1024
NUM_EXPERTS = 8
TOP_K = 2

TOK_BLK = 512


def _router_body(x_ref, gw_ref, logits_ref, coef_ref):
    x = x_ref[...]
    gw = gw_ref[...]
    logits = jax.lax.dot_general(
        x, gw, (((1,), (1,)), ((), ())), preferred_element_type=jnp.float32
    )
    logits_ref[...] = logits
    m = jnp.max(logits, axis=-1, keepdims=True)
    ex = jnp.exp(logits - m)
    probs = ex / jnp.sum(ex, axis=-1, keepdims=True)
    eids = jax.lax.broadcasted_iota(jnp.int32, probs.shape, 1)
    i1 = jnp.argmax(probs, axis=-1, keepdims=True)
    v1 = jnp.max(probs, axis=-1, keepdims=True)
    masked = jnp.where(eids == i1, -1.0, probs)
    i2 = jnp.argmax(masked, axis=-1, keepdims=True)
    v2 = jnp.max(masked, axis=-1, keepdims=True)
    s = v1 + v2
    coef = jnp.where(eids == i1, v1 / s, 0.0) + jnp.where(eids == i2, v2 / s, 0.0)
    coef_ref[...] = coef


def _expert_body(x_ref, gu_ref, dn_ref, coef_ref, out_ref):
    e = pl.program_id(1)
    x = x_ref[...]
    gu_w = gu_ref[0]
    dn_w = dn_ref[0]
    gu = jax.lax.dot_general(
        x, gu_w, (((1,), (1,)), ((), ())), preferred_element_type=jnp.float32
    )
    gate = gu[:, :INTER]
    up = gu[:, INTER:]
    act = gate * jax.lax.logistic(gate) * up
    eo = jax.lax.dot_general(
        act, dn_w, (((1,), (1,)), ((), ())), preferred_element_type=jnp.float32
    )
    coef = coef_ref[...]
    eids = jax.lax.broadcasted_iota(jnp.int32, coef.shape, 1)
    col = jnp.sum(jnp.where(eids == e, coef, 0.0), axis=-1, keepdims=True)
    contrib = eo * col

    @pl.when(e == 0)
    def _():
        out_ref[...] = contrib

    @pl.when(e != 0)
    def _():
        out_ref[...] += contrib


@functools.partial(jax.jit, static_argnames=())
def kernel(x, gate_weight, gate_up_proj, down_proj):
    Bv, Sv, H = x.shape
    T = Bv * Sv
    x_flat = x.reshape(T, H)
    nblk = T // TOK_BLK

    logits, coef = pl.pallas_call(
        _router_body,
        grid=(nblk,),
        in_specs=[
            pl.BlockSpec((TOK_BLK, H), lambda b: (b, 0)),
            pl.BlockSpec((NUM_EXPERTS, H), lambda b: (0, 0)),
        ],
        out_specs=[
            pl.BlockSpec((TOK_BLK, NUM_EXPERTS), lambda b: (b, 0)),
            pl.BlockSpec((TOK_BLK, NUM_EXPERTS), lambda b: (b, 0)),
        ],
        out_shape=[
            jax.ShapeDtypeStruct((T, NUM_EXPERTS), jnp.float32),
            jax.ShapeDtypeStruct((T, NUM_EXPERTS), jnp.float32),
        ],
    )(x_flat, gate_weight)

    out = pl.pallas_call(
        _expert_body,
        grid=(nblk, NUM_EXPERTS),
        in_specs=[
            pl.BlockSpec((TOK_BLK, H), lambda b, e: (b, 0)),
            pl.BlockSpec((1, 2 * INTER, H), lambda b, e: (e, 0, 0)),
            pl.BlockSpec((1, H, INTER), lambda b, e: (e, 0, 0)),
            pl.BlockSpec((TOK_BLK, NUM_EXPERTS), lambda b, e: (b, 0)),
        ],
        out_specs=pl.BlockSpec((TOK_BLK, H), lambda b, e: (b, 0)),
        out_shape=jax.ShapeDtypeStruct((T, H), jnp.float32),
    )(x_flat, gate_up_proj, down_proj, coef)

    return out.reshape(Bv, Sv, H), logits


# speedup vs baseline: 1.7199x; 1.1463x over previous
import functools

import jax
import jax.numpy as jnp
from jax import lax
from jax.experimental import pallas as pl
from jax.experimental.pallas import tpu as pltpu
from jax.experimental.pallas import tpu_sc as plsc

HIDDEN = 1024
INTER = 1024
NUM_EXPERTS = 8
TOP_K = 2

T_TOKENS = 4096
NPAIRS = T_TOKENS * TOP_K
M_BLK = 512
N_BLOCKS = NPAIRS // M_BLK
N_STEPS = N_BLOCKS + NUM_EXPERTS - 1

NW = 32
PAIRS_PER_W = NPAIRS // NW
GROUPS_TOTAL = NPAIRS // 16
GROUPS_PER_W = PAIRS_PER_W // 16
ROW_CHUNK = 64

def _sc_mesh():
    return plsc.VectorSubcoreMesh(core_axis_name="c", subcore_axis_name="s")


def _router_body(x_ref, gw_ref, logits_ref, e1_ref, e2_ref, w1_ref, w2_ref):
    x = x_ref[...]
    gw = gw_ref[...]
    logits = lax.dot_general(
        x, gw, (((1,), (1,)), ((), ())), preferred_element_type=jnp.float32
    )
    logits_ref[...] = logits
    m = jnp.max(logits, axis=-1, keepdims=True)
    ex = jnp.exp(logits - m)
    probs = ex / jnp.sum(ex, axis=-1, keepdims=True)
    eids = lax.broadcasted_iota(jnp.int32, probs.shape, 1)
    i1 = jnp.argmax(probs, axis=-1, keepdims=True)
    v1 = jnp.max(probs, axis=-1, keepdims=True)
    masked = jnp.where(eids == i1, -1.0, probs)
    i2 = jnp.argmax(masked, axis=-1, keepdims=True)
    v2 = jnp.max(masked, axis=-1, keepdims=True)
    s = v1 + v2
    e1_ref[...] = i1.astype(jnp.int32)
    e2_ref[...] = i2.astype(jnp.int32)
    w1_ref[...] = v1 / s
    w2_ref[...] = v2 / s


def _router(x_flat, gate_weight):
    nblk = T_TOKENS // M_BLK
    outs = pl.pallas_call(
        _router_body,
        grid=(nblk,),
        in_specs=[
            pl.BlockSpec((M_BLK, HIDDEN), lambda b: (b, 0)),
            pl.BlockSpec((NUM_EXPERTS, HIDDEN), lambda b: (0, 0)),
        ],
        out_specs=[
            pl.BlockSpec((M_BLK, NUM_EXPERTS), lambda b: (b, 0)),
            pl.BlockSpec((M_BLK, 1), lambda b: (b, 0)),
            pl.BlockSpec((M_BLK, 1), lambda b: (b, 0)),
            pl.BlockSpec((M_BLK, 1), lambda b: (b, 0)),
            pl.BlockSpec((M_BLK, 1), lambda b: (b, 0)),
        ],
        out_shape=[
            jax.ShapeDtypeStruct((T_TOKENS, NUM_EXPERTS), jnp.float32),
            jax.ShapeDtypeStruct((T_TOKENS, 1), jnp.int32),
            jax.ShapeDtypeStruct((T_TOKENS, 1), jnp.int32),
            jax.ShapeDtypeStruct((T_TOKENS, 1), jnp.float32),
            jax.ShapeDtypeStruct((T_TOKENS, 1), jnp.float32),
        ],
    )(x_flat, gate_weight)
    return outs


_GDN = lax.GatherDimensionNumbers(
    offset_dims=(), collapsed_slice_dims=(0,), start_index_map=(0,))


def _xlane(vec, idxvec):
    return lax.gather(vec, idxvec.reshape(16, 1), _GDN, (1,),
                      mode=lax.GatherScatterMode.PROMISE_IN_BOUNDS)


def _splat(vec, j, lane):
    return _xlane(vec, lane * 0 + j)


def _total_splat(v, lane):
    for k in (1, 2, 4, 8):
        v = v + _xlane(v, lane ^ k)
    return v


def _cumsum16(v, lane):
    for k in (1, 2, 4, 8):
        sh = _xlane(v, jnp.maximum(lane - k, 0))
        v = v + jnp.where(lane >= k, sh, 0)
    return v


def _dispatch_body(eflat_hbm, x_hbm, xs_hbm, rank_hbm, counts_hbm,
                   ids_v, rank_v, rank2d_v, tok_v, rowbuf_v, counts_v,
                   sem1, sem2):
    wid = lax.axis_index("s") * 2 + lax.axis_index("c")
    lane = lax.iota(jnp.int32, 16)
    pltpu.sync_copy(eflat_hbm, ids_v)
    g_lo = wid * GROUPS_PER_W
    zero = jnp.zeros((16,), jnp.int32)

    prefix = []
    totals = []
    for e in range(NUM_EXPERTS):
        def body_e(g, pc, e=e):
            v = ids_v[pl.ds(g * 16, 16)]
            return pc + 1 - jnp.minimum(jnp.abs(v - e), 1)

        p_pre = lax.fori_loop(0, g_lo, body_e, zero)
        prefix.append(_total_splat(p_pre, lane))
        p_tot = lax.fori_loop(g_lo, GROUPS_TOTAL, body_e, p_pre)
        totals.append(_total_splat(p_tot, lane))

    offs = []
    acc = zero
    for e in range(NUM_EXPERTS):
        offs.append(acc)
        acc = acc + totals[e]
    start = [offs[e] + prefix[e] for e in range(NUM_EXPERTS)]

    for g in range(GROUPS_PER_W):
        v = ids_v[pl.ds((g_lo + g) * 16, 16)]
        rank_vec = jnp.zeros((16,), jnp.int32)
        for e in range(NUM_EXPERTS):
            mi = 1 - jnp.minimum(jnp.abs(v - e), 1)
            incl = _cumsum16(mi, lane)
            rank_vec = rank_vec + mi * (start[e] + incl - 1 - rank_vec)
            start[e] = start[e] + _splat(incl, 15, lane)
        rank_v[pl.ds(g * 16, 16)] = rank_vec
        rank2d_v[g // 4, pl.ds((g % 4) * 16, 16)] = rank_vec

    pltpu.sync_copy(rank_v, rank_hbm.at[pl.ds(wid * PAIRS_PER_W, PAIRS_PER_W)])

    cvec = jnp.zeros((16,), jnp.int32)
    for e in range(NUM_EXPERTS):
        cvec = jnp.where(lane == e, totals[e], cvec)

    @pl.when(wid == 0)
    def _():
        counts_v[...] = cvec
        pltpu.sync_copy(counts_v, counts_hbm)

    base = wid * PAIRS_PER_W
    for c in range(PAIRS_PER_W // ROW_CHUNK):
        for g in range(ROW_CHUNK // 16):
            pv = base + c * ROW_CHUNK + g * 16 + lane
            tok_v[pl.ds(g * 16, 16)] = pv >> 1
        pltpu.async_copy(x_hbm.at[tok_v], rowbuf_v, sem1).wait()
        pltpu.async_copy(rowbuf_v, xs_hbm.at[rank2d_v.at[c]], sem2).wait()


def _dispatch(eflat, x_flat):
    f = pl.kernel(
        _dispatch_body,
        mesh=_sc_mesh(),
        out_type=[
            jax.ShapeDtypeStruct((NPAIRS, HIDDEN), jnp.float32),
            jax.ShapeDtypeStruct((NPAIRS,), jnp.int32),
            jax.ShapeDtypeStruct((16,), jnp.int32),
        ],
        scratch_types=[
            pltpu.VMEM((NPAIRS,), jnp.int32),
            pltpu.VMEM((PAIRS_PER_W,), jnp.int32),
            pltpu.VMEM((PAIRS_PER_W // ROW_CHUNK, ROW_CHUNK), jnp.int32),
            pltpu.VMEM((ROW_CHUNK,), jnp.int32),
            pltpu.VMEM((ROW_CHUNK, HIDDEN), jnp.float32),
            pltpu.VMEM((16,), jnp.int32),
            pltpu.SemaphoreType.DMA,
            pltpu.SemaphoreType.DMA,
        ],
    )
    return f(eflat, x_flat)


def _gmm_body(blk_s, exp_s, lo_s, hi_s, first_s, xs_ref, gu_ref, dn_ref,
              out_ref):
    t = pl.program_id(0)
    x = xs_ref[...]
    gu_w = gu_ref[0]
    dn_w = dn_ref[0]
    gu = lax.dot_general(
        x, gu_w, (((1,), (1,)), ((), ())), preferred_element_type=jnp.float32
    )
    gate = gu[:, :INTER]
    up = gu[:, INTER:]
    act = gate * lax.logistic(gate) * up
    eo = lax.dot_general(
        act, dn_w, (((1,), (1,)), ((), ())), preferred_element_type=jnp.float32
    )
    rows = blk_s[t] * M_BLK + lax.broadcasted_iota(jnp.int32, (M_BLK, 1), 0)
    mask = (rows >= lo_s[t]) & (rows < hi_s[t])
    contrib = jnp.where(mask, eo, 0.0)

    @pl.when(first_s[t] == 1)
    def _():
        out_ref[...] = contrib

    @pl.when(first_s[t] == 0)
    def _():
        out_ref[...] += contrib


def _schedule(counts16):
    counts = counts16[:NUM_EXPERTS]
    offs = jnp.concatenate(
        [jnp.zeros((1,), jnp.int32), jnp.cumsum(counts, dtype=jnp.int32)]
    )
    lo_e = offs[:NUM_EXPERTS]
    hi_e = offs[1:NUM_EXPERTS + 1]
    blk_f = jnp.repeat(jnp.arange(N_BLOCKS, dtype=jnp.int32), NUM_EXPERTS)
    exp_f = jnp.tile(jnp.arange(NUM_EXPERTS, dtype=jnp.int32), N_BLOCKS)
    blo = blk_f * M_BLK
    bhi = blo + M_BLK
    active = (lo_e[exp_f] < bhi) & (hi_e[exp_f] > blo)
    pos = jnp.cumsum(active.astype(jnp.int32)) - 1
    first2d = jnp.cumsum(active.reshape(N_BLOCKS, NUM_EXPERTS), axis=1) == 1
    first_f = (active & first2d.reshape(-1)).astype(jnp.int32)
    tgt = jnp.where(active, pos, N_STEPS)

    def scat(default, vals):
        out = jnp.full((N_STEPS,), default, jnp.int32)
        return out.at[tgt].set(vals.astype(jnp.int32), mode="drop")

    blk_arr = scat(N_BLOCKS - 1, blk_f)
    exp_arr = scat(NUM_EXPERTS - 1, exp_f)
    lo_arr = scat(0, lo_e[exp_f])
    hi_arr = scat(0, hi_e[exp_f])
    first_arr = scat(0, first_f)
    return blk_arr, exp_arr, lo_arr, hi_arr, first_arr


def _gmm(sched, xs, gate_up_proj, down_proj):
    blk_arr, exp_arr, lo_arr, hi_arr, first_arr = sched
    grid_spec = pltpu.PrefetchScalarGridSpec(
        num_scalar_prefetch=5,
        grid=(N_STEPS,),
        in_specs=[
            pl.BlockSpec((M_BLK, HIDDEN),
                         lambda t, blk, exp, lo, hi, first: (blk[t], 0)),
            pl.BlockSpec((1, 2 * INTER, HIDDEN),
                         lambda t, blk, exp, lo, hi, first: (exp[t], 0, 0)),
            pl.BlockSpec((1, HIDDEN, INTER),
                         lambda t, blk, exp, lo, hi, first: (exp[t], 0, 0)),
        ],
        out_specs=pl.BlockSpec((M_BLK, HIDDEN),
                               lambda t, blk, exp, lo, hi, first: (blk[t], 0)),
    )
    return pl.pallas_call(
        _gmm_body,
        grid_spec=grid_spec,
        out_shape=jax.ShapeDtypeStruct((NPAIRS, HIDDEN), jnp.float32),
    )(blk_arr, exp_arr, lo_arr, hi_arr, first_arr,
      xs, gate_up_proj, down_proj)


def _combine_body(os_hbm, rank_hbm, w_hbm, fin_hbm,
                  rk_v, w_v, buf_v, out_v, sem1):
    wid = lax.axis_index("s") * 2 + lax.axis_index("c")
    lane = lax.iota(jnp.int32, 16)
    pltpu.sync_copy(rank_hbm.at[pl.ds(wid * PAIRS_PER_W, PAIRS_PER_W)], rk_v)
    pltpu.sync_copy(w_hbm.at[pl.ds(wid * PAIRS_PER_W, PAIRS_PER_W)], w_v)
    tok0 = wid * (PAIRS_PER_W // 2)
    for c in range(PAIRS_PER_W // ROW_CHUNK):
        pltpu.async_copy(os_hbm.at[rk_v.at[pl.ds(c * ROW_CHUNK, ROW_CHUNK)]],
                         buf_v, sem1).wait()
        for i in range(ROW_CHUNK // 2):
            wv = w_v[pl.ds(c * ROW_CHUNK + (i // 8) * 16, 16)]
            j0 = 2 * (i % 8)
            w0 = _splat(wv, j0, lane)
            w1 = _splat(wv, j0 + 1, lane)

            def jbody(j, _, i=i, w0=w0, w1=w1):
                a = buf_v[2 * i, pl.ds(j * 16, 16)]
                b = buf_v[2 * i + 1, pl.ds(j * 16, 16)]
                out_v[i, pl.ds(j * 16, 16)] = a * w0 + b * w1
                return 0

            lax.fori_loop(0, HIDDEN // 16, jbody, 0)
        pltpu.sync_copy(
            out_v,
            fin_hbm.at[pl.ds(tok0 + c * (ROW_CHUNK // 2), ROW_CHUNK // 2), :])


def _combine(out_sorted, rank, wflat):
    f = pl.kernel(
        _combine_body,
        mesh=_sc_mesh(),
        out_type=jax.ShapeDtypeStruct((T_TOKENS, HIDDEN), jnp.float32),
        scratch_types=[
            pltpu.VMEM((PAIRS_PER_W,), jnp.int32),
            pltpu.VMEM((PAIRS_PER_W,), jnp.float32),
            pltpu.VMEM((ROW_CHUNK, HIDDEN), jnp.float32),
            pltpu.VMEM((ROW_CHUNK // 2, HIDDEN), jnp.float32),
            pltpu.SemaphoreType.DMA,
        ],
    )
    return f(out_sorted, rank, wflat)


@functools.partial(jax.jit)
def kernel(x, gate_weight, gate_up_proj, down_proj):
    Bv, Sv, H = x.shape
    x_flat = x.reshape(Bv * Sv, H)
    logits, e1, e2, w1, w2 = _router(x_flat, gate_weight)
    eflat = jnp.concatenate([e1, e2], axis=1).reshape(-1)
    wflat = jnp.concatenate([w1, w2], axis=1).reshape(-1)
    xs, rank, counts16 = _dispatch(eflat, x_flat)
    sched = _schedule(counts16)
    out_sorted = _gmm(sched, xs, gate_up_proj, down_proj)
    final = _combine(out_sorted, rank, wflat)
    return final.reshape(Bv, Sv, H), logits
